# baseline (speedup 1.0000x reference)
import jax
import jax.numpy as jnp
from jax.experimental import pallas as pl
from jax.experimental.pallas import tpu as pltpu

_N_TARGETS = 6
_R = 5
_H = 512
_W = 512
_AH = _H - _R + 1
_HP = _H + 32
_NQ = 4


def _iota(shape, dim):
    return jax.lax.broadcasted_iota(jnp.int32, shape, dim)


def _hsum(t):
    acc = t
    n = t.shape[0]
    for k in range(1, _R):
        acc = acc + jnp.concatenate(
            [t[:, k:], jnp.zeros((n, k), jnp.float32)], axis=1)
    return acc


def _rowstats(aggblk, colia):
    m = jnp.max(aggblk, axis=1, keepdims=True)
    cc = jnp.min(jnp.where(aggblk == m, colia, jnp.int32(_W)),
                 axis=1, keepdims=True)
    return m, cc


_flat = None


def _peaks_kernel(hm_ref, out_ref, hm_s, h_s, rmax_s, rcol_s):
    colia = _iota((_AH, _W), 1)
    flat = _iota((_H, _W), 0) * _W + _iota((_H, _W), 1)
    big = jnp.int32(_H * _W)
    rowi16 = _iota((16, _W), 0)
    coli16 = _iota((16, _W), 1)
    rio = _iota((_H, 1), 0)
    rio16 = _iota((16, 1), 0)
    cio = _iota((1, _W), 1)

    def init(q):
        hm0 = hm_ref[q]
        hm_s[q * _H:(q + 1) * _H, :] = hm0
        h0 = _hsum(hm0)
        h_s[q * _HP:q * _HP + _H, :] = h0
        h_s[q * _HP + _H:(q + 1) * _HP, :] = jnp.zeros(
            (_HP - _H, _W), jnp.float32)
        v = h0[0:_AH, :]
        for k in range(1, _R):
            v = v + h0[k:k + _AH, :]
        agg0 = jnp.where(colia < _AH, v / float(_R * _R), -1.0)
        m0, c0 = _rowstats(agg0, colia)
        rmax_s[q * _H:q * _H + _AH, :] = m0
        rmax_s[q * _H + _AH:(q + 1) * _H, :] = jnp.full(
            (_H - _AH, 1), -1.0, jnp.float32)
        rcol_s[q * _H:q * _H + _AH, :] = c0
        rcol_s[q * _H + _AH:(q + 1) * _H, :] = jnp.zeros(
            (_H - _AH, 1), jnp.int32)

    for q in range(_NQ):
        init(q)

    def one(q, i, res):
        hb = q * _H
        rm = rmax_s[hb:hb + _H, :]
        gmax = jnp.max(rm)
        r = jnp.min(jnp.where(rm == gmax, rio, jnp.int32(_H)))
        c = jnp.min(jnp.where(rio == r, rcol_s[hb:hb + _H, :],
                              jnp.int32(_W)))

        rs = pl.multiple_of(hb + jnp.minimum((r // 8) * 8, _H - 16), 8)
        tile = hm_s[pl.ds(rs, 16), :]
        rr = r + hb - rs
        inwin = (((rio16 >= rr) & (rio16 < rr + _R))
                 & ((cio >= c) & (cio < c + _R)))
        mval = jnp.max(jnp.where(inwin, tile, -1.0))
        conf = jnp.sum(jnp.where(inwin, tile, 0.0))

        hm = hm_s[hb:hb + _H, :]
        fi2 = jnp.min(jnp.where(hm == mval, flat, big))
        rh = fi2 // _W
        ch = jax.lax.rem(fi2, _W)

        ztile = jnp.where(inwin, 0.0, tile)
        hm_s[pl.ds(rs, 16), :] = ztile
        h_s[pl.ds(pl.multiple_of(rs + q * (_HP - _H), 8), 16), :] = (
            _hsum(ztile))

        rs3 = pl.multiple_of(
            q * _HP + jnp.minimum(
                jnp.maximum(((r - (_R - 1)) // 8) * 8, 0), _H - 16), 8)
        h32 = h_s[pl.ds(rs3, 32), :]
        acc = h32[0:16]
        for k in range(1, _R):
            acc = acc + h32[k:k + 16]
        aggblk = jnp.where(((rs3 - q * _HP) + rio16 < _AH) & (cio < _AH),
                           acc / float(_R * _R), -1.0)
        mb, cb = _rowstats(aggblk, coli16)
        rmb = pl.multiple_of(rs3 - q * _HP + hb, 8)
        rmax_s[pl.ds(rmb, 16), :] = mb
        rcol_s[pl.ds(rmb, 16), :] = cb

        sel = _iota((8, 128), 0) == i
        coli8 = _iota((8, 128), 1)
        res = jnp.where(sel & (coli8 == 0), rh.astype(jnp.float32), res)
        res = jnp.where(sel & (coli8 == 1), ch.astype(jnp.float32), res)
        res = jnp.where(sel & (coli8 == 2), conf, res)
        return res

    def body(i, carry):
        return tuple(one(q, i, carry[q]) for q in range(_NQ))

    res = jax.lax.fori_loop(
        0, _N_TARGETS, body,
        tuple(jnp.zeros((8, 128), jnp.float32) for _ in range(_NQ)))
    for q in range(_NQ):
        out_ref[q] = res[q]


def kernel(heatmap):
    hm = heatmap[:, 0]
    b = hm.shape[0]
    out = pl.pallas_call(
        _peaks_kernel,
        grid=(b // _NQ,),
        in_specs=[pl.BlockSpec((_NQ, _H, _W), lambda i: (i, 0, 0))],
        out_specs=pl.BlockSpec((_NQ, 8, 128), lambda i: (i, 0, 0)),
        out_shape=jax.ShapeDtypeStruct((b, 8, 128), jnp.float32),
        scratch_shapes=[
            pltpu.VMEM((_NQ * _H, _W), jnp.float32),
            pltpu.VMEM((_NQ * _HP, _W), jnp.float32),
            pltpu.VMEM((_NQ * _H, 1), jnp.float32),
            pltpu.VMEM((_NQ * _H, 1), jnp.int32),
        ],
        compiler_params=pltpu.CompilerParams(
            dimension_semantics=("parallel",),
        ),
    )(hm)
    end_points = out[:, :_N_TARGETS, 0:2]
    confidences = out[:, :_N_TARGETS, 2]
    return end_points, confidences


# speedup vs baseline: 1.0083x; 1.0083x over previous
import jax
import jax.numpy as jnp
from jax.experimental import pallas as pl
from jax.experimental.pallas import tpu as pltpu

_N_TARGETS = 6
_R = 5
_H = 512
_W = 512
_AH = _H - _R + 1
_HP = _H + 32
_NQ = 4


def _iota(shape, dim):
    return jax.lax.broadcasted_iota(jnp.int32, shape, dim)


def _hsum(t):
    acc = t
    n = t.shape[0]
    for k in range(1, _R):
        acc = acc + jnp.concatenate(
            [t[:, k:], jnp.zeros((n, k), jnp.float32)], axis=1)
    return acc


def _rowstats(aggblk, colia):
    m = jnp.max(aggblk, axis=1, keepdims=True)
    cc = jnp.min(jnp.where(aggblk == m, colia, jnp.int32(_W)),
                 axis=1, keepdims=True)
    return m, cc


_flat = None


def _peaks_kernel(hm_ref, out_ref, hm_s, h_s, rmax_s, rcol_s):
    colia = _iota((_AH, _W), 1)
    flat = _iota((_H, _W), 0) * _W + _iota((_H, _W), 1)
    big = jnp.int32(_H * _W)
    rowi16 = _iota((16, _W), 0)
    coli16 = _iota((16, _W), 1)
    rio = _iota((_H, 1), 0)

    def init(q):
        hm0 = hm_ref[q]
        hm_s[q * _H:(q + 1) * _H, :] = hm0
        h0 = _hsum(hm0)
        h_s[q * _HP:q * _HP + _H, :] = h0
        h_s[q * _HP + _H:(q + 1) * _HP, :] = jnp.zeros(
            (_HP - _H, _W), jnp.float32)
        v = h0[0:_AH, :]
        for k in range(1, _R):
            v = v + h0[k:k + _AH, :]
        agg0 = jnp.where(colia < _AH, v / float(_R * _R), -1.0)
        m0, c0 = _rowstats(agg0, colia)
        rmax_s[q * _H:q * _H + _AH, :] = m0
        rmax_s[q * _H + _AH:(q + 1) * _H, :] = jnp.full(
            (_H - _AH, 1), -1.0, jnp.float32)
        rcol_s[q * _H:q * _H + _AH, :] = c0
        rcol_s[q * _H + _AH:(q + 1) * _H, :] = jnp.zeros(
            (_H - _AH, 1), jnp.int32)

    for q in range(_NQ):
        init(q)

    def one(q, i, res):
        hb = q * _H
        rm = rmax_s[hb:hb + _H, :]
        gmax = jnp.max(rm)
        r = jnp.min(jnp.where(rm == gmax, rio, jnp.int32(_H)))
        c = jnp.min(jnp.where(rio == r, rcol_s[hb:hb + _H, :],
                              jnp.int32(_W)))

        rs = pl.multiple_of(hb + jnp.minimum((r // 8) * 8, _H - 16), 8)
        tile = hm_s[pl.ds(rs, 16), :]
        rr = r + hb - rs
        inwin = ((rowi16 >= rr) & (rowi16 < rr + _R)
                 & (coli16 >= c) & (coli16 < c + _R))
        mval = jnp.max(jnp.where(inwin, tile, -1.0))
        conf = jnp.sum(jnp.where(inwin, tile, 0.0))

        hm = hm_s[hb:hb + _H, :]
        fi2 = jnp.min(jnp.where(hm == mval, flat, big))
        rh = fi2 // _W
        ch = jax.lax.rem(fi2, _W)

        ztile = jnp.where(inwin, 0.0, tile)
        hm_s[pl.ds(rs, 16), :] = ztile
        h_s[pl.ds(pl.multiple_of(rs + q * (_HP - _H), 8), 16), :] = (
            _hsum(ztile))

        rs3 = pl.multiple_of(
            q * _HP + jnp.minimum(
                jnp.maximum(((r - (_R - 1)) // 8) * 8, 0), _H - 16), 8)
        h32 = h_s[pl.ds(rs3, 32), :]
        acc = h32[0:16]
        for k in range(1, _R):
            acc = acc + h32[k:k + 16]
        rowabs16 = (rs3 - q * _HP) + rowi16
        aggblk = jnp.where((rowabs16 < _AH) & (coli16 < _AH),
                           acc / float(_R * _R), -1.0)
        mb, cb = _rowstats(aggblk, coli16)
        rmb = pl.multiple_of(rs3 - q * _HP + hb, 8)
        rmax_s[pl.ds(rmb, 16), :] = mb
        rcol_s[pl.ds(rmb, 16), :] = cb

        sel = _iota((8, 128), 0) == i
        coli8 = _iota((8, 128), 1)
        res = jnp.where(sel & (coli8 == 0), rh.astype(jnp.float32), res)
        res = jnp.where(sel & (coli8 == 1), ch.astype(jnp.float32), res)
        res = jnp.where(sel & (coli8 == 2), conf, res)
        return res

    def body(i, carry):
        return tuple(one(q, i, carry[q]) for q in range(_NQ))

    res = jax.lax.fori_loop(
        0, _N_TARGETS, body,
        tuple(jnp.zeros((8, 128), jnp.float32) for _ in range(_NQ)))
    for q in range(_NQ):
        out_ref[q] = res[q]


def kernel(heatmap):
    hm = heatmap[:, 0]
    b = hm.shape[0]
    out = pl.pallas_call(
        _peaks_kernel,
        grid=(b // _NQ,),
        in_specs=[pl.BlockSpec((_NQ, _H, _W), lambda i: (i, 0, 0))],
        out_specs=pl.BlockSpec((_NQ, 8, 128), lambda i: (i, 0, 0)),
        out_shape=jax.ShapeDtypeStruct((b, 8, 128), jnp.float32),
        scratch_shapes=[
            pltpu.VMEM((_NQ * _H, _W), jnp.float32),
            pltpu.VMEM((_NQ * _HP, _W), jnp.float32),
            pltpu.VMEM((_NQ * _H, 1), jnp.float32),
            pltpu.VMEM((_NQ * _H, 1), jnp.int32),
        ],
        compiler_params=pltpu.CompilerParams(
            dimension_semantics=("parallel",),
        ),
    )(hm)
    end_points = out[:, :_N_TARGETS, 0:2]
    confidences = out[:, :_N_TARGETS, 2]
    return end_points, confidences


# speedup vs baseline: 1.0118x; 1.0035x over previous
import jax
import jax.numpy as jnp
from jax.experimental import pallas as pl
from jax.experimental.pallas import tpu as pltpu

_N_TARGETS = 6
_R = 5
_H = 512
_W = 512
_AH = _H - _R + 1
_HP = _H + 32
_NQ = 8


def _iota(shape, dim):
    return jax.lax.broadcasted_iota(jnp.int32, shape, dim)


def _hsum(t):
    acc = t
    n = t.shape[0]
    for k in range(1, _R):
        acc = acc + jnp.concatenate(
            [t[:, k:], jnp.zeros((n, k), jnp.float32)], axis=1)
    return acc


def _rowstats(aggblk, colia):
    m = jnp.max(aggblk, axis=1, keepdims=True)
    cc = jnp.min(jnp.where(aggblk == m, colia, jnp.int32(_W)),
                 axis=1, keepdims=True)
    return m, cc


_flat = None


def _peaks_kernel(hm_ref, out_ref, hm_s, h_s, rmax_s, rcol_s):
    colia = _iota((_AH, _W), 1)
    flat = _iota((_H, _W), 0) * _W + _iota((_H, _W), 1)
    big = jnp.int32(_H * _W)
    rowi16 = _iota((16, _W), 0)
    coli16 = _iota((16, _W), 1)
    rio = _iota((_H, 1), 0)

    def init(q):
        hm0 = hm_ref[q]
        hm_s[q * _H:(q + 1) * _H, :] = hm0
        h0 = _hsum(hm0)
        h_s[q * _HP:q * _HP + _H, :] = h0
        h_s[q * _HP + _H:(q + 1) * _HP, :] = jnp.zeros(
            (_HP - _H, _W), jnp.float32)
        v = h0[0:_AH, :]
        for k in range(1, _R):
            v = v + h0[k:k + _AH, :]
        agg0 = jnp.where(colia < _AH, v / float(_R * _R), -1.0)
        m0, c0 = _rowstats(agg0, colia)
        rmax_s[q * _H:q * _H + _AH, :] = m0
        rmax_s[q * _H + _AH:(q + 1) * _H, :] = jnp.full(
            (_H - _AH, 1), -1.0, jnp.float32)
        rcol_s[q * _H:q * _H + _AH, :] = c0
        rcol_s[q * _H + _AH:(q + 1) * _H, :] = jnp.zeros(
            (_H - _AH, 1), jnp.int32)

    for q in range(_NQ):
        init(q)

    def one(q, i, res):
        hb = q * _H
        rm = rmax_s[hb:hb + _H, :]
        gmax = jnp.max(rm)
        r = jnp.min(jnp.where(rm == gmax, rio, jnp.int32(_H)))
        c = jnp.min(jnp.where(rio == r, rcol_s[hb:hb + _H, :],
                              jnp.int32(_W)))

        rs = pl.multiple_of(hb + jnp.minimum((r // 8) * 8, _H - 16), 8)
        tile = hm_s[pl.ds(rs, 16), :]
        rr = r + hb - rs
        inwin = ((rowi16 >= rr) & (rowi16 < rr + _R)
                 & (coli16 >= c) & (coli16 < c + _R))
        mval = jnp.max(jnp.where(inwin, tile, -1.0))
        conf = jnp.sum(jnp.where(inwin, tile, 0.0))

        hm = hm_s[hb:hb + _H, :]
        fi2 = jnp.min(jnp.where(hm == mval, flat, big))
        rh = fi2 // _W
        ch = jax.lax.rem(fi2, _W)

        ztile = jnp.where(inwin, 0.0, tile)
        hm_s[pl.ds(rs, 16), :] = ztile
        h_s[pl.ds(pl.multiple_of(rs + q * (_HP - _H), 8), 16), :] = (
            _hsum(ztile))

        rs3 = pl.multiple_of(
            q * _HP + jnp.minimum(
                jnp.maximum(((r - (_R - 1)) // 8) * 8, 0), _H - 16), 8)
        h32 = h_s[pl.ds(rs3, 32), :]
        acc = h32[0:16]
        for k in range(1, _R):
            acc = acc + h32[k:k + 16]
        rowabs16 = (rs3 - q * _HP) + rowi16
        aggblk = jnp.where((rowabs16 < _AH) & (coli16 < _AH),
                           acc / float(_R * _R), -1.0)
        mb, cb = _rowstats(aggblk, coli16)
        rmb = pl.multiple_of(rs3 - q * _HP + hb, 8)
        rmax_s[pl.ds(rmb, 16), :] = mb
        rcol_s[pl.ds(rmb, 16), :] = cb

        sel = _iota((8, 128), 0) == i
        coli8 = _iota((8, 128), 1)
        res = jnp.where(sel & (coli8 == 0), rh.astype(jnp.float32), res)
        res = jnp.where(sel & (coli8 == 1), ch.astype(jnp.float32), res)
        res = jnp.where(sel & (coli8 == 2), conf, res)
        return res

    def body(i, carry):
        return tuple(one(q, i, carry[q]) for q in range(_NQ))

    res = jax.lax.fori_loop(
        0, _N_TARGETS, body,
        tuple(jnp.zeros((8, 128), jnp.float32) for _ in range(_NQ)))
    for q in range(_NQ):
        out_ref[q] = res[q]


def kernel(heatmap):
    hm = heatmap[:, 0]
    b = hm.shape[0]
    out = pl.pallas_call(
        _peaks_kernel,
        grid=(b // _NQ,),
        in_specs=[pl.BlockSpec((_NQ, _H, _W), lambda i: (i, 0, 0))],
        out_specs=pl.BlockSpec((_NQ, 8, 128), lambda i: (i, 0, 0)),
        out_shape=jax.ShapeDtypeStruct((b, 8, 128), jnp.float32),
        scratch_shapes=[
            pltpu.VMEM((_NQ * _H, _W), jnp.float32),
            pltpu.VMEM((_NQ * _HP, _W), jnp.float32),
            pltpu.VMEM((_NQ * _H, 1), jnp.float32),
            pltpu.VMEM((_NQ * _H, 1), jnp.int32),
        ],
        compiler_params=pltpu.CompilerParams(
            dimension_semantics=("parallel",),
        ),
    )(hm)
    end_points = out[:, :_N_TARGETS, 0:2]
    confidences = out[:, :_N_TARGETS, 2]
    return end_points, confidences
